# Initial kernel scaffold; baseline (speedup 1.0000x reference)
import functools

import jax
import jax.numpy as jnp
from jax import lax
from jax.experimental import pallas as pl
from jax.experimental.pallas import tpu as pltpu
from jax.experimental.pallas import tpu_sc as plsc

f32 = jnp.float32
i32 = jnp.int32

N = 10000
E = 320000
F = 128
H = 64
G = 64
C = 10

NP = 10240
CHUNK = 128
CPT = 80
NCORES = 2
NSUB = 16
TILES = NCORES * NSUB
EP = TILES * CPT * CHUNK
ROWS_PT = NP // NSUB
RB = 1024


def _zero_zbuf(zbuf, width):
    def body(r, _):
        for j in range(width // 16):
            zbuf[r, pl.ds(j * 16, 16)] = jnp.zeros((16,), f32)
        return 0
    lax.fori_loop(0, 16, body, 0)


def _zero_acc_slice(zbuf, acc, s):
    base = s * ROWS_PT
    def body(j, _):
        pltpu.sync_copy(zbuf, acc.at[pl.ds(base + j * 16, 16)])
        return 0
    lax.fori_loop(0, ROWS_PT // 16, body, 0)


def _msg_body(g_hbm, src_hbm, dst_hbm, out_hbm, sidx, didx, rows, zbuf, acc, sem):
    c = lax.axis_index("c")
    s = lax.axis_index("s")
    wid = c * NSUB + s
    _zero_zbuf(zbuf, H)
    _zero_acc_slice(zbuf, acc, s)
    pltpu.sync_copy(src_hbm.at[pl.ds(wid * CPT, CPT)], sidx)
    pltpu.sync_copy(dst_hbm.at[pl.ds(wid * CPT, CPT)], didx)
    plsc.subcore_barrier()

    def step(ci, _):
        pltpu.async_copy(g_hbm.at[sidx.at[ci]], rows, sem).wait()
        pltpu.sync_copy(rows, acc.at[didx.at[ci]], add=True)
        return 0
    lax.fori_loop(0, CPT, step, 0)

    plsc.subcore_barrier()
    base = s * ROWS_PT
    pltpu.sync_copy(acc.at[pl.ds(base, ROWS_PT)],
                    out_hbm.at[c, pl.ds(base, ROWS_PT)])


def _deg_body(dst_hbm, out_hbm, didx, ones_v, zbuf, acc, sem):
    c = lax.axis_index("c")
    s = lax.axis_index("s")
    wid = c * NSUB + s
    def fill_ones(r, _):
        ones_v[r, :] = jnp.ones((16,), f32)
        return 0
    lax.fori_loop(0, CHUNK, fill_ones, 0)
    _zero_zbuf(zbuf, 16)
    _zero_acc_slice(zbuf, acc, s)
    pltpu.sync_copy(dst_hbm.at[pl.ds(wid * CPT, CPT)], didx)
    plsc.subcore_barrier()

    def step(ci, _):
        pltpu.sync_copy(ones_v, acc.at[didx.at[ci]], add=True)
        return 0
    lax.fori_loop(0, CPT, step, 0)

    plsc.subcore_barrier()
    base = s * ROWS_PT
    pltpu.sync_copy(acc.at[pl.ds(base, ROWS_PT)],
                    out_hbm.at[c, pl.ds(base, ROWS_PT)])


_SC_MESH = plsc.VectorSubcoreMesh(core_axis_name="c", subcore_axis_name="s")

_msg_call = functools.partial(
    pl.kernel,
    mesh=_SC_MESH,
    out_type=jax.ShapeDtypeStruct((NCORES, NP, H), f32),
    scratch_types=[
        pltpu.VMEM((CPT, CHUNK), i32),
        pltpu.VMEM((CPT, CHUNK), i32),
        pltpu.VMEM((CHUNK, H), f32),
        pltpu.VMEM((16, H), f32),
        pltpu.VMEM_SHARED((NP, H), f32),
        pltpu.SemaphoreType.DMA,
    ],
)(_msg_body)

_deg_call = functools.partial(
    pl.kernel,
    mesh=_SC_MESH,
    out_type=jax.ShapeDtypeStruct((NCORES, NP, 16), f32),
    scratch_types=[
        pltpu.VMEM((CPT, CHUNK), i32),
        pltpu.VMEM((CHUNK, 16), f32),
        pltpu.VMEM((16, 16), f32),
        pltpu.VMEM_SHARED((NP, 16), f32),
        pltpu.SemaphoreType.DMA,
    ],
)(_deg_body)


def _mm_body(x_ref, w_ref, o_ref):
    o_ref[...] = jnp.dot(x_ref[...], w_ref[...], preferred_element_type=f32)


_mm = pl.pallas_call(
    _mm_body,
    grid=(NP // RB,),
    in_specs=[pl.BlockSpec((RB, F), lambda i: (i, 0)),
              pl.BlockSpec((F, H), lambda i: (0, 0))],
    out_specs=pl.BlockSpec((RB, H), lambda i: (i, 0)),
    out_shape=jax.ShapeDtypeStruct((NP, H), f32),
)


def _dinv_g_body(degA_ref, degB_ref, h_ref, dinv_ref, g_ref):
    deg = degA_ref[:, :1] + degB_ref[:, :1] + 1.0
    dinv = lax.rsqrt(deg)
    dinv_ref[...] = dinv
    g_ref[...] = h_ref[...] * dinv


_dinv_g = pl.pallas_call(
    _dinv_g_body,
    grid=(NP // RB,),
    in_specs=[pl.BlockSpec((RB, 16), lambda i: (i, 0)),
              pl.BlockSpec((RB, 16), lambda i: (i, 0)),
              pl.BlockSpec((RB, H), lambda i: (i, 0))],
    out_specs=[pl.BlockSpec((RB, 1), lambda i: (i, 0)),
               pl.BlockSpec((RB, H), lambda i: (i, 0))],
    out_shape=[jax.ShapeDtypeStruct((NP, 1), f32),
               jax.ShapeDtypeStruct((NP, H), f32)],
)


def _layer_body(accA_ref, accB_ref, g_ref, dinv_ref, b_ref, w_ref, go_ref):
    dinv = dinv_ref[...]
    h = dinv * (accA_ref[...] + accB_ref[...] + g_ref[...]) + b_ref[...]
    h = jnp.maximum(h, 0.0)
    go_ref[...] = dinv * jnp.dot(h, w_ref[...], preferred_element_type=f32)


_layer = pl.pallas_call(
    _layer_body,
    grid=(NP // RB,),
    in_specs=[pl.BlockSpec((RB, H), lambda i: (i, 0)),
              pl.BlockSpec((RB, H), lambda i: (i, 0)),
              pl.BlockSpec((RB, H), lambda i: (i, 0)),
              pl.BlockSpec((RB, 1), lambda i: (i, 0)),
              pl.BlockSpec((1, H), lambda i: (0, 0)),
              pl.BlockSpec((H, H), lambda i: (0, 0))],
    out_specs=pl.BlockSpec((RB, H), lambda i: (i, 0)),
    out_shape=jax.ShapeDtypeStruct((NP, H), f32),
)


def _pool_body(accA_ref, accB_ref, g_ref, dinv_ref, b_ref, batch_ref,
               wc1_ref, bc1_ref, wc2_ref, bc2_ref, out_ref, seg_acc, cnt_acc):
    i = pl.program_id(0)
    dinv = dinv_ref[...]
    h = dinv * (accA_ref[...] + accB_ref[...] + g_ref[...]) + b_ref[...]
    h = jnp.maximum(h, 0.0)
    row = lax.broadcasted_iota(f32, (RB, 1), 0) + i * RB
    valid = row < float(N)
    seg = lax.broadcasted_iota(f32, (1, G), 1)
    onehot = jnp.where((batch_ref[...] == seg) & valid, 1.0, 0.0)
    contrib = lax.dot_general(onehot, h, (((0,), (0,)), ((), ())),
                              preferred_element_type=f32)
    csum = lax.dot_general(onehot, jnp.ones((RB, 1), f32),
                           (((0,), (0,)), ((), ())),
                           preferred_element_type=f32)

    @pl.when(i == 0)
    def _():
        seg_acc[...] = contrib
        cnt_acc[...] = csum
        out_ref[...] = jnp.zeros((G, C), f32)

    @pl.when(i > 0)
    def _():
        seg_acc[...] += contrib
        cnt_acc[...] += csum

    @pl.when(i == NP // RB - 1)
    def _():
        pooled = seg_acc[...] / jnp.maximum(cnt_acc[...], 1.0)
        z = jnp.maximum(
            jnp.dot(pooled, wc1_ref[...], preferred_element_type=f32)
            + bc1_ref[...], 0.0)
        out_ref[...] = (jnp.dot(z, wc2_ref[...], preferred_element_type=f32)
                        + bc2_ref[...])


_pool = pl.pallas_call(
    _pool_body,
    grid=(NP // RB,),
    in_specs=[pl.BlockSpec((RB, H), lambda i: (i, 0)),
              pl.BlockSpec((RB, H), lambda i: (i, 0)),
              pl.BlockSpec((RB, H), lambda i: (i, 0)),
              pl.BlockSpec((RB, 1), lambda i: (i, 0)),
              pl.BlockSpec((1, H), lambda i: (0, 0)),
              pl.BlockSpec((RB, 1), lambda i: (i, 0)),
              pl.BlockSpec((H, 32), lambda i: (0, 0)),
              pl.BlockSpec((1, 32), lambda i: (0, 0)),
              pl.BlockSpec((32, C), lambda i: (0, 0)),
              pl.BlockSpec((1, C), lambda i: (0, 0))],
    out_specs=pl.BlockSpec((G, C), lambda i: (0, 0)),
    out_shape=jax.ShapeDtypeStruct((G, C), f32),
    scratch_shapes=[pltpu.VMEM((G, H), f32), pltpu.VMEM((G, 1), f32)],
)


def kernel(x, edge_index, batch, W1, b1, W2, b2, W3, b3, Wc1, bc1, Wc2, bc2):
    src = edge_index[0].astype(i32)
    dst = edge_index[1].astype(i32)
    src_p = jnp.concatenate(
        [src, jnp.zeros((EP - E,), i32)]).reshape(TILES * CPT, CHUNK)
    dst_p = jnp.concatenate(
        [dst, jnp.full((EP - E,), N, i32)]).reshape(TILES * CPT, CHUNK)
    x_p = jnp.pad(x, ((0, NP - N), (0, 0)))
    batch_p = jnp.pad(batch.astype(f32), (0, NP - N)).reshape(NP, 1)

    deg2 = _deg_call(dst_p)
    h1pre = _mm(x_p, W1)
    dinv, g1 = _dinv_g(deg2[0], deg2[1], h1pre)
    a1 = _msg_call(g1, src_p, dst_p)
    g2 = _layer(a1[0], a1[1], g1, dinv, b1.reshape(1, H), W2)
    a2 = _msg_call(g2, src_p, dst_p)
    g3 = _layer(a2[0], a2[1], g2, dinv, b2.reshape(1, H), W3)
    a3 = _msg_call(g3, src_p, dst_p)
    out = _pool(a3[0], a3[1], g3, dinv, b3.reshape(1, H), batch_p,
                Wc1, bc1.reshape(1, 32), Wc2, bc2.reshape(1, C))
    return out


# speedup vs baseline: 10.8282x; 10.8282x over previous
import functools

import jax
import jax.numpy as jnp
from jax import lax
from jax.experimental import pallas as pl
from jax.experimental.pallas import tpu as pltpu
from jax.experimental.pallas import tpu_sc as plsc

f32 = jnp.float32
i32 = jnp.int32

N = 10000
E = 320000
F = 128
H = 64
G = 64
C = 10

NP = 10240
CHUNK = 128
CPT = 80
NCORES = 2
NSUB = 16
TILES = NCORES * NSUB
EP = TILES * CPT * CHUNK
ROWS_PT = NP // NSUB
RB = 1024


def _zero_zbuf(zbuf, width):
    def body(r, _):
        for j in range(width // 16):
            zbuf[r, pl.ds(j * 16, 16)] = jnp.zeros((16,), f32)
        return 0
    lax.fori_loop(0, 16, body, 0)


def _zero_acc_slice(zbuf, acc, s):
    base = s * ROWS_PT
    def body(j, _):
        pltpu.sync_copy(zbuf, acc.at[pl.ds(base + j * 16, 16)])
        return 0
    lax.fori_loop(0, ROWS_PT // 16, body, 0)


def _msg_body(g_hbm, src_hbm, dst_hbm, out_hbm, sidx, didx, rows, zbuf, acc, sem):
    c = lax.axis_index("c")
    s = lax.axis_index("s")
    wid = c * NSUB + s
    _zero_zbuf(zbuf, H)
    _zero_acc_slice(zbuf, acc, s)
    pltpu.sync_copy(src_hbm.at[pl.ds(wid * CPT, CPT)], sidx)
    pltpu.sync_copy(dst_hbm.at[pl.ds(wid * CPT, CPT)], didx)
    plsc.subcore_barrier()

    def step(ci, _):
        pltpu.async_copy(g_hbm.at[sidx.at[ci]], rows, sem).wait()
        pltpu.sync_copy(rows, acc.at[didx.at[ci]], add=True)
        return 0
    lax.fori_loop(0, CPT, step, 0)

    plsc.subcore_barrier()
    base = s * ROWS_PT
    pltpu.sync_copy(acc.at[pl.ds(base, ROWS_PT)],
                    out_hbm.at[c, pl.ds(base, ROWS_PT)])


def _deg_body(dst_hbm, out_hbm, didx, ones_v, zbuf, acc, sem):
    c = lax.axis_index("c")
    s = lax.axis_index("s")
    wid = c * NSUB + s
    def fill_ones(r, _):
        ones_v[r, :] = jnp.ones((16,), f32)
        return 0
    lax.fori_loop(0, CHUNK, fill_ones, 0)
    _zero_zbuf(zbuf, 16)
    _zero_acc_slice(zbuf, acc, s)
    pltpu.sync_copy(dst_hbm.at[pl.ds(wid * CPT, CPT)], didx)
    plsc.subcore_barrier()

    def step(ci, _):
        pltpu.sync_copy(ones_v, acc.at[didx.at[ci]], add=True)
        return 0
    lax.fori_loop(0, CPT, step, 0)

    plsc.subcore_barrier()
    base = s * ROWS_PT
    pltpu.sync_copy(acc.at[pl.ds(base, ROWS_PT)],
                    out_hbm.at[c, pl.ds(base, ROWS_PT)])


_SC_MESH = plsc.VectorSubcoreMesh(core_axis_name="c", subcore_axis_name="s")

_msg_call = functools.partial(
    pl.kernel,
    mesh=_SC_MESH,
    compiler_params=pltpu.CompilerParams(use_tc_tiling_on_sc=False),
    out_type=jax.ShapeDtypeStruct((NCORES, NP, H), f32),
    scratch_types=[
        pltpu.VMEM((CPT, CHUNK), i32),
        pltpu.VMEM((CPT, CHUNK), i32),
        pltpu.VMEM((CHUNK, H), f32),
        pltpu.VMEM((16, H), f32),
        pltpu.VMEM_SHARED((NP, H), f32),
        pltpu.SemaphoreType.DMA,
    ],
)(_msg_body)

_deg_call = functools.partial(
    pl.kernel,
    mesh=_SC_MESH,
    compiler_params=pltpu.CompilerParams(use_tc_tiling_on_sc=False),
    out_type=jax.ShapeDtypeStruct((NCORES, NP, 16), f32),
    scratch_types=[
        pltpu.VMEM((CPT, CHUNK), i32),
        pltpu.VMEM((CHUNK, 16), f32),
        pltpu.VMEM((16, 16), f32),
        pltpu.VMEM_SHARED((NP, 16), f32),
        pltpu.SemaphoreType.DMA,
    ],
)(_deg_body)


def _mm_body(x_ref, w_ref, o_ref):
    o_ref[...] = jnp.dot(x_ref[...], w_ref[...], preferred_element_type=f32)


_mm = pl.pallas_call(
    _mm_body,
    grid=(NP // RB,),
    in_specs=[pl.BlockSpec((RB, F), lambda i: (i, 0)),
              pl.BlockSpec((F, H), lambda i: (0, 0))],
    out_specs=pl.BlockSpec((RB, H), lambda i: (i, 0)),
    out_shape=jax.ShapeDtypeStruct((NP, H), f32),
)


def _dinv_g_body(degA_ref, degB_ref, h_ref, dinv_ref, g_ref):
    deg = degA_ref[:, :1] + degB_ref[:, :1] + 1.0
    dinv = lax.rsqrt(deg)
    dinv_ref[...] = dinv
    g_ref[...] = h_ref[...] * dinv


_dinv_g = pl.pallas_call(
    _dinv_g_body,
    grid=(NP // RB,),
    in_specs=[pl.BlockSpec((RB, 16), lambda i: (i, 0)),
              pl.BlockSpec((RB, 16), lambda i: (i, 0)),
              pl.BlockSpec((RB, H), lambda i: (i, 0))],
    out_specs=[pl.BlockSpec((RB, 1), lambda i: (i, 0)),
               pl.BlockSpec((RB, H), lambda i: (i, 0))],
    out_shape=[jax.ShapeDtypeStruct((NP, 1), f32),
               jax.ShapeDtypeStruct((NP, H), f32)],
)


def _layer_body(accA_ref, accB_ref, g_ref, dinv_ref, b_ref, w_ref, go_ref):
    dinv = dinv_ref[...]
    h = dinv * (accA_ref[...] + accB_ref[...] + g_ref[...]) + b_ref[...]
    h = jnp.maximum(h, 0.0)
    go_ref[...] = dinv * jnp.dot(h, w_ref[...], preferred_element_type=f32)


_layer = pl.pallas_call(
    _layer_body,
    grid=(NP // RB,),
    in_specs=[pl.BlockSpec((RB, H), lambda i: (i, 0)),
              pl.BlockSpec((RB, H), lambda i: (i, 0)),
              pl.BlockSpec((RB, H), lambda i: (i, 0)),
              pl.BlockSpec((RB, 1), lambda i: (i, 0)),
              pl.BlockSpec((1, H), lambda i: (0, 0)),
              pl.BlockSpec((H, H), lambda i: (0, 0))],
    out_specs=pl.BlockSpec((RB, H), lambda i: (i, 0)),
    out_shape=jax.ShapeDtypeStruct((NP, H), f32),
)


def _pool_body(accA_ref, accB_ref, g_ref, dinv_ref, b_ref, batch_ref,
               wc1_ref, bc1_ref, wc2_ref, bc2_ref, out_ref, seg_acc, cnt_acc):
    i = pl.program_id(0)
    dinv = dinv_ref[...]
    h = dinv * (accA_ref[...] + accB_ref[...] + g_ref[...]) + b_ref[...]
    h = jnp.maximum(h, 0.0)
    row = lax.broadcasted_iota(i32, (RB, 1), 0) + i * RB
    valid = row < N
    seg = lax.broadcasted_iota(i32, (1, G), 1)
    batch_i = batch_ref[...].astype(i32)
    onehot = jnp.where((batch_i == seg) & valid, 1.0, 0.0)
    contrib = lax.dot_general(onehot, h, (((0,), (0,)), ((), ())),
                              preferred_element_type=f32)
    csum = lax.dot_general(onehot, jnp.ones((RB, 1), f32),
                           (((0,), (0,)), ((), ())),
                           preferred_element_type=f32)

    @pl.when(i == 0)
    def _():
        seg_acc[...] = contrib
        cnt_acc[...] = csum
        out_ref[...] = jnp.zeros((G, C), f32)

    @pl.when(i > 0)
    def _():
        seg_acc[...] += contrib
        cnt_acc[...] += csum

    @pl.when(i == NP // RB - 1)
    def _():
        pooled = seg_acc[...] / jnp.maximum(cnt_acc[...], 1.0)
        z = jnp.maximum(
            jnp.dot(pooled, wc1_ref[...], preferred_element_type=f32)
            + bc1_ref[...], 0.0)
        out_ref[...] = (jnp.dot(z, wc2_ref[...], preferred_element_type=f32)
                        + bc2_ref[...])


_pool = pl.pallas_call(
    _pool_body,
    grid=(NP // RB,),
    in_specs=[pl.BlockSpec((RB, H), lambda i: (i, 0)),
              pl.BlockSpec((RB, H), lambda i: (i, 0)),
              pl.BlockSpec((RB, H), lambda i: (i, 0)),
              pl.BlockSpec((RB, 1), lambda i: (i, 0)),
              pl.BlockSpec((1, H), lambda i: (0, 0)),
              pl.BlockSpec((RB, 1), lambda i: (i, 0)),
              pl.BlockSpec((H, 32), lambda i: (0, 0)),
              pl.BlockSpec((1, 32), lambda i: (0, 0)),
              pl.BlockSpec((32, C), lambda i: (0, 0)),
              pl.BlockSpec((1, C), lambda i: (0, 0))],
    out_specs=pl.BlockSpec((G, C), lambda i: (0, 0)),
    out_shape=jax.ShapeDtypeStruct((G, C), f32),
    scratch_shapes=[pltpu.VMEM((G, H), f32), pltpu.VMEM((G, 1), f32)],
)


def kernel(x, edge_index, batch, W1, b1, W2, b2, W3, b3, Wc1, bc1, Wc2, bc2):
    src = edge_index[0].astype(i32)
    dst = edge_index[1].astype(i32)
    src_p = jnp.concatenate(
        [src, jnp.zeros((EP - E,), i32)]).reshape(TILES * CPT, CHUNK)
    dst_p = jnp.concatenate(
        [dst, jnp.full((EP - E,), N, i32)]).reshape(TILES * CPT, CHUNK)
    x_p = jnp.pad(x, ((0, NP - N), (0, 0)))
    batch_p = jnp.pad(batch.astype(f32), (0, NP - N)).reshape(NP, 1)

    deg2 = _deg_call(dst_p)
    h1pre = _mm(x_p, W1)
    dinv, g1 = _dinv_g(deg2[0], deg2[1], h1pre)
    a1 = _msg_call(g1, src_p, dst_p)
    g2 = _layer(a1[0], a1[1], g1, dinv, b1.reshape(1, H), W2)
    a2 = _msg_call(g2, src_p, dst_p)
    g3 = _layer(a2[0], a2[1], g2, dinv, b2.reshape(1, H), W3)
    a3 = _msg_call(g3, src_p, dst_p)
    out = _pool(a3[0], a3[1], g3, dinv, b3.reshape(1, H), batch_p,
                Wc1, bc1.reshape(1, 32), Wc2, bc2.reshape(1, C))
    return out


# speedup vs baseline: 12.3629x; 1.1417x over previous
import functools

import jax
import jax.numpy as jnp
from jax import lax
from jax.experimental import pallas as pl
from jax.experimental.pallas import tpu as pltpu
from jax.experimental.pallas import tpu_sc as plsc

f32 = jnp.float32
i32 = jnp.int32

N = 10000
E = 320000
F = 128
H = 64
G = 64
C = 10

NP = 10240
CHUNK = 128
CPT = 80
NCORES = 2
NSUB = 16
TILES = NCORES * NSUB
EP = TILES * CPT * CHUNK
ROWS_PT = NP // NSUB
RB = 1024


def _zero_zbuf(zbuf, width):
    def body(r, _):
        for j in range(width // 16):
            zbuf[r, pl.ds(j * 16, 16)] = jnp.zeros((16,), f32)
        return 0
    lax.fori_loop(0, 16, body, 0)


def _zero_acc_slice(zbuf, acc, s):
    base = s * ROWS_PT
    def body(j, _):
        pltpu.sync_copy(zbuf, acc.at[pl.ds(base + j * 16, 16)])
        return 0
    lax.fori_loop(0, ROWS_PT // 16, body, 0)


NBUF = 4


def _msg_body(g_hbm, src_hbm, dst_hbm, out_hbm, sidx, didx, rows, zbuf, acc,
              sem0, sem1, sem2, sem3):
    c = lax.axis_index("c")
    s = lax.axis_index("s")
    wid = c * NSUB + s
    sems = (sem0, sem1, sem2, sem3)
    _zero_zbuf(zbuf, H)
    _zero_acc_slice(zbuf, acc, s)
    pltpu.sync_copy(src_hbm.at[pl.ds(wid * CPT, CPT)], sidx)
    pltpu.sync_copy(dst_hbm.at[pl.ds(wid * CPT, CPT)], didx)
    plsc.subcore_barrier()

    for b in range(NBUF):
        pltpu.async_copy(g_hbm.at[sidx.at[b]], rows.at[b], sems[b])

    def step(ci4, _):
        for b in range(NBUF):
            ci = ci4 * NBUF + b
            pltpu.make_async_copy(g_hbm.at[sidx.at[ci]], rows.at[b],
                                  sems[b]).wait()
            pltpu.sync_copy(rows.at[b], acc.at[didx.at[ci]], add=True)
            pltpu.async_copy(g_hbm.at[sidx.at[ci + NBUF]], rows.at[b], sems[b])
        return 0
    lax.fori_loop(0, CPT // NBUF - 1, step, 0)
    for b in range(NBUF):
        ci = CPT - NBUF + b
        pltpu.make_async_copy(g_hbm.at[sidx.at[ci]], rows.at[b],
                              sems[b]).wait()
        pltpu.sync_copy(rows.at[b], acc.at[didx.at[ci]], add=True)

    plsc.subcore_barrier()
    base = s * ROWS_PT
    pltpu.sync_copy(acc.at[pl.ds(base, ROWS_PT)],
                    out_hbm.at[c, pl.ds(base, ROWS_PT)])


def _deg_body(dst_hbm, out_hbm, didx, ones_v, zbuf, acc, sem):
    c = lax.axis_index("c")
    s = lax.axis_index("s")
    wid = c * NSUB + s
    def fill_ones(r, _):
        ones_v[r, :] = jnp.ones((16,), f32)
        return 0
    lax.fori_loop(0, CHUNK, fill_ones, 0)
    _zero_zbuf(zbuf, 16)
    _zero_acc_slice(zbuf, acc, s)
    pltpu.sync_copy(dst_hbm.at[pl.ds(wid * CPT, CPT)], didx)
    plsc.subcore_barrier()

    def step(ci, _):
        pltpu.sync_copy(ones_v, acc.at[didx.at[ci]], add=True)
        return 0
    lax.fori_loop(0, CPT, step, 0)

    plsc.subcore_barrier()
    base = s * ROWS_PT
    pltpu.sync_copy(acc.at[pl.ds(base, ROWS_PT)],
                    out_hbm.at[c, pl.ds(base, ROWS_PT)])


_SC_MESH = plsc.VectorSubcoreMesh(core_axis_name="c", subcore_axis_name="s")

_msg_call = functools.partial(
    pl.kernel,
    mesh=_SC_MESH,
    compiler_params=pltpu.CompilerParams(use_tc_tiling_on_sc=False),
    out_type=jax.ShapeDtypeStruct((NCORES, NP, H), f32),
    scratch_types=[
        pltpu.VMEM((CPT, CHUNK), i32),
        pltpu.VMEM((CPT, CHUNK), i32),
        pltpu.VMEM((NBUF, CHUNK, H), f32),
        pltpu.VMEM((16, H), f32),
        pltpu.VMEM_SHARED((NP, H), f32),
        pltpu.SemaphoreType.DMA,
        pltpu.SemaphoreType.DMA,
        pltpu.SemaphoreType.DMA,
        pltpu.SemaphoreType.DMA,
    ],
)(_msg_body)

_deg_call = functools.partial(
    pl.kernel,
    mesh=_SC_MESH,
    compiler_params=pltpu.CompilerParams(use_tc_tiling_on_sc=False),
    out_type=jax.ShapeDtypeStruct((NCORES, NP, 16), f32),
    scratch_types=[
        pltpu.VMEM((CPT, CHUNK), i32),
        pltpu.VMEM((CHUNK, 16), f32),
        pltpu.VMEM((16, 16), f32),
        pltpu.VMEM_SHARED((NP, 16), f32),
        pltpu.SemaphoreType.DMA,
    ],
)(_deg_body)


def _mm_body(x_ref, w_ref, o_ref):
    o_ref[...] = jnp.dot(x_ref[...], w_ref[...], preferred_element_type=f32)


_mm = pl.pallas_call(
    _mm_body,
    grid=(NP // RB,),
    in_specs=[pl.BlockSpec((RB, F), lambda i: (i, 0)),
              pl.BlockSpec((F, H), lambda i: (0, 0))],
    out_specs=pl.BlockSpec((RB, H), lambda i: (i, 0)),
    out_shape=jax.ShapeDtypeStruct((NP, H), f32),
)


def _dinv_g_body(degA_ref, degB_ref, h_ref, dinv_ref, g_ref):
    deg = degA_ref[:, :1] + degB_ref[:, :1] + 1.0
    dinv = lax.rsqrt(deg)
    dinv_ref[...] = dinv
    g_ref[...] = h_ref[...] * dinv


_dinv_g = pl.pallas_call(
    _dinv_g_body,
    grid=(NP // RB,),
    in_specs=[pl.BlockSpec((RB, 16), lambda i: (i, 0)),
              pl.BlockSpec((RB, 16), lambda i: (i, 0)),
              pl.BlockSpec((RB, H), lambda i: (i, 0))],
    out_specs=[pl.BlockSpec((RB, 1), lambda i: (i, 0)),
               pl.BlockSpec((RB, H), lambda i: (i, 0))],
    out_shape=[jax.ShapeDtypeStruct((NP, 1), f32),
               jax.ShapeDtypeStruct((NP, H), f32)],
)


def _layer_body(accA_ref, accB_ref, g_ref, dinv_ref, b_ref, w_ref, go_ref):
    dinv = dinv_ref[...]
    h = dinv * (accA_ref[...] + accB_ref[...] + g_ref[...]) + b_ref[...]
    h = jnp.maximum(h, 0.0)
    go_ref[...] = dinv * jnp.dot(h, w_ref[...], preferred_element_type=f32)


_layer = pl.pallas_call(
    _layer_body,
    grid=(NP // RB,),
    in_specs=[pl.BlockSpec((RB, H), lambda i: (i, 0)),
              pl.BlockSpec((RB, H), lambda i: (i, 0)),
              pl.BlockSpec((RB, H), lambda i: (i, 0)),
              pl.BlockSpec((RB, 1), lambda i: (i, 0)),
              pl.BlockSpec((1, H), lambda i: (0, 0)),
              pl.BlockSpec((H, H), lambda i: (0, 0))],
    out_specs=pl.BlockSpec((RB, H), lambda i: (i, 0)),
    out_shape=jax.ShapeDtypeStruct((NP, H), f32),
)


def _pool_body(accA_ref, accB_ref, g_ref, dinv_ref, b_ref, batch_ref,
               wc1_ref, bc1_ref, wc2_ref, bc2_ref, out_ref, seg_acc, cnt_acc):
    i = pl.program_id(0)
    dinv = dinv_ref[...]
    h = dinv * (accA_ref[...] + accB_ref[...] + g_ref[...]) + b_ref[...]
    h = jnp.maximum(h, 0.0)
    row = lax.broadcasted_iota(i32, (RB, 1), 0) + i * RB
    valid = row < N
    seg = lax.broadcasted_iota(i32, (1, G), 1)
    batch_i = batch_ref[...].astype(i32)
    onehot = jnp.where((batch_i == seg) & valid, 1.0, 0.0)
    contrib = lax.dot_general(onehot, h, (((0,), (0,)), ((), ())),
                              preferred_element_type=f32)
    csum = lax.dot_general(onehot, jnp.ones((RB, 1), f32),
                           (((0,), (0,)), ((), ())),
                           preferred_element_type=f32)

    @pl.when(i == 0)
    def _():
        seg_acc[...] = contrib
        cnt_acc[...] = csum
        out_ref[...] = jnp.zeros((G, C), f32)

    @pl.when(i > 0)
    def _():
        seg_acc[...] += contrib
        cnt_acc[...] += csum

    @pl.when(i == NP // RB - 1)
    def _():
        pooled = seg_acc[...] / jnp.maximum(cnt_acc[...], 1.0)
        z = jnp.maximum(
            jnp.dot(pooled, wc1_ref[...], preferred_element_type=f32)
            + bc1_ref[...], 0.0)
        out_ref[...] = (jnp.dot(z, wc2_ref[...], preferred_element_type=f32)
                        + bc2_ref[...])


_pool = pl.pallas_call(
    _pool_body,
    grid=(NP // RB,),
    in_specs=[pl.BlockSpec((RB, H), lambda i: (i, 0)),
              pl.BlockSpec((RB, H), lambda i: (i, 0)),
              pl.BlockSpec((RB, H), lambda i: (i, 0)),
              pl.BlockSpec((RB, 1), lambda i: (i, 0)),
              pl.BlockSpec((1, H), lambda i: (0, 0)),
              pl.BlockSpec((RB, 1), lambda i: (i, 0)),
              pl.BlockSpec((H, 32), lambda i: (0, 0)),
              pl.BlockSpec((1, 32), lambda i: (0, 0)),
              pl.BlockSpec((32, C), lambda i: (0, 0)),
              pl.BlockSpec((1, C), lambda i: (0, 0))],
    out_specs=pl.BlockSpec((G, C), lambda i: (0, 0)),
    out_shape=jax.ShapeDtypeStruct((G, C), f32),
    scratch_shapes=[pltpu.VMEM((G, H), f32), pltpu.VMEM((G, 1), f32)],
)


def kernel(x, edge_index, batch, W1, b1, W2, b2, W3, b3, Wc1, bc1, Wc2, bc2):
    src = edge_index[0].astype(i32)
    dst = edge_index[1].astype(i32)
    src_p = jnp.concatenate(
        [src, jnp.zeros((EP - E,), i32)]).reshape(TILES * CPT, CHUNK)
    dst_p = jnp.concatenate(
        [dst, jnp.full((EP - E,), N, i32)]).reshape(TILES * CPT, CHUNK)
    x_p = jnp.pad(x, ((0, NP - N), (0, 0)))
    batch_p = jnp.pad(batch.astype(f32), (0, NP - N)).reshape(NP, 1)

    deg2 = _deg_call(dst_p)
    h1pre = _mm(x_p, W1)
    dinv, g1 = _dinv_g(deg2[0], deg2[1], h1pre)
    a1 = _msg_call(g1, src_p, dst_p)
    g2 = _layer(a1[0], a1[1], g1, dinv, b1.reshape(1, H), W2)
    a2 = _msg_call(g2, src_p, dst_p)
    g3 = _layer(a2[0], a2[1], g2, dinv, b2.reshape(1, H), W3)
    a3 = _msg_call(g3, src_p, dst_p)
    out = _pool(a3[0], a3[1], g3, dinv, b3.reshape(1, H), batch_p,
                Wc1, bc1.reshape(1, 32), Wc2, bc2.reshape(1, C))
    return out


# speedup vs baseline: 36.0585x; 2.9167x over previous
import functools

import jax
import jax.numpy as jnp
from jax import lax
from jax.experimental import pallas as pl
from jax.experimental.pallas import tpu as pltpu
from jax.experimental.pallas import tpu_sc as plsc

f32 = jnp.float32
i32 = jnp.int32

N = 10000
E = 320000
F = 128
H = 64
G = 64
C = 10

NP = 10240
CHUNK = 128
CPT = 80
NCORES = 2
NSUB = 16
TILES = NCORES * NSUB
EP = TILES * CPT * CHUNK
ROWS_PT = NP // NSUB
RB = 1024


def _zero_zbuf(zbuf, width):
    def body(r, _):
        for j in range(width // 16):
            zbuf[r, pl.ds(j * 16, 16)] = jnp.zeros((16,), f32)
        return 0
    lax.fori_loop(0, 16, body, 0)


def _zero_acc_slice(zbuf, acc, s):
    base = s * ROWS_PT
    def body(j, _):
        pltpu.sync_copy(zbuf, acc.at[pl.ds(base + j * 16, 16)])
        return 0
    lax.fori_loop(0, ROWS_PT // 16, body, 0)


NBUF = 4


def _msg_body(g_hbm, src_hbm, dst_hbm, out_hbm, sidx, didx, rows, zbuf, acc,
              sem0, sem1, sem2, sem3):
    c = lax.axis_index("c")
    s = lax.axis_index("s")
    wid = c * NSUB + s
    sems = (sem0, sem1, sem2, sem3)
    _zero_zbuf(zbuf, H)
    _zero_acc_slice(zbuf, acc, s)
    pltpu.sync_copy(src_hbm.at[pl.ds(wid * CPT, CPT)], sidx)
    pltpu.sync_copy(dst_hbm.at[pl.ds(wid * CPT, CPT)], didx)
    plsc.subcore_barrier()

    for b in range(NBUF):
        pltpu.async_copy(g_hbm.at[sidx.at[b]], rows.at[b], sems[b])

    def step(ci4, _):
        for b in range(NBUF):
            ci = ci4 * NBUF + b
            pltpu.make_async_copy(g_hbm.at[sidx.at[ci]], rows.at[b],
                                  sems[b]).wait()
            pltpu.sync_copy(rows.at[b], acc.at[didx.at[ci]], add=True)
            pltpu.async_copy(g_hbm.at[sidx.at[ci + NBUF]], rows.at[b], sems[b])
        return 0
    lax.fori_loop(0, CPT // NBUF - 1, step, 0)
    for b in range(NBUF):
        ci = CPT - NBUF + b
        pltpu.make_async_copy(g_hbm.at[sidx.at[ci]], rows.at[b],
                              sems[b]).wait()
        pltpu.sync_copy(rows.at[b], acc.at[didx.at[ci]], add=True)

    plsc.subcore_barrier()
    base = s * ROWS_PT
    pltpu.sync_copy(acc.at[pl.ds(base, ROWS_PT)],
                    out_hbm.at[c, pl.ds(base, ROWS_PT)])


def _deg_body(dst_hbm, out_hbm, didx, ones_v, zbuf, acc, sem):
    c = lax.axis_index("c")
    s = lax.axis_index("s")
    wid = c * NSUB + s
    def fill_ones(r, _):
        ones_v[r, :] = jnp.ones((16,), f32)
        return 0
    lax.fori_loop(0, CHUNK, fill_ones, 0)
    _zero_zbuf(zbuf, 16)
    _zero_acc_slice(zbuf, acc, s)
    pltpu.sync_copy(dst_hbm.at[pl.ds(wid * CPT, CPT)], didx)
    plsc.subcore_barrier()

    def step(ci, _):
        pltpu.sync_copy(ones_v, acc.at[didx.at[ci]], add=True)
        return 0
    lax.fori_loop(0, CPT, step, 0)

    plsc.subcore_barrier()
    base = s * ROWS_PT
    pltpu.sync_copy(acc.at[pl.ds(base, ROWS_PT)],
                    out_hbm.at[c, pl.ds(base, ROWS_PT)])


_SC_MESH = plsc.VectorSubcoreMesh(core_axis_name="c", subcore_axis_name="s")

_msg_call = functools.partial(
    pl.kernel,
    mesh=_SC_MESH,
    compiler_params=pltpu.CompilerParams(use_tc_tiling_on_sc=False),
    out_type=jax.ShapeDtypeStruct((NCORES, NP, H), f32),
    scratch_types=[
        pltpu.VMEM((CPT, CHUNK), i32),
        pltpu.VMEM((CPT, CHUNK), i32),
        pltpu.VMEM((NBUF, CHUNK, H), f32),
        pltpu.VMEM((16, H), f32),
        pltpu.VMEM_SHARED((NP, H), f32),
        pltpu.SemaphoreType.DMA,
        pltpu.SemaphoreType.DMA,
        pltpu.SemaphoreType.DMA,
        pltpu.SemaphoreType.DMA,
    ],
)(_msg_body)

_deg_call = functools.partial(
    pl.kernel,
    mesh=_SC_MESH,
    compiler_params=pltpu.CompilerParams(use_tc_tiling_on_sc=False),
    out_type=jax.ShapeDtypeStruct((NCORES, NP, 16), f32),
    scratch_types=[
        pltpu.VMEM((CPT, CHUNK), i32),
        pltpu.VMEM((CHUNK, 16), f32),
        pltpu.VMEM((16, 16), f32),
        pltpu.VMEM_SHARED((NP, 16), f32),
        pltpu.SemaphoreType.DMA,
    ],
)(_deg_body)


def _mm_body(x_ref, w_ref, o_ref):
    o_ref[...] = jnp.dot(x_ref[...], w_ref[...], preferred_element_type=f32)


_mm = pl.pallas_call(
    _mm_body,
    grid=(NP // RB,),
    in_specs=[pl.BlockSpec((RB, F), lambda i: (i, 0)),
              pl.BlockSpec((F, H), lambda i: (0, 0))],
    out_specs=pl.BlockSpec((RB, H), lambda i: (i, 0)),
    out_shape=jax.ShapeDtypeStruct((NP, H), f32),
)


def _dinv_g_body(degA_ref, degB_ref, h_ref, dinv_ref, g_ref):
    deg = degA_ref[:, :1] + degB_ref[:, :1] + 1.0
    dinv = lax.rsqrt(deg)
    dinv_ref[...] = dinv
    g_ref[...] = h_ref[...] * dinv


_dinv_g = pl.pallas_call(
    _dinv_g_body,
    grid=(NP // RB,),
    in_specs=[pl.BlockSpec((RB, 16), lambda i: (i, 0)),
              pl.BlockSpec((RB, 16), lambda i: (i, 0)),
              pl.BlockSpec((RB, H), lambda i: (i, 0))],
    out_specs=[pl.BlockSpec((RB, 1), lambda i: (i, 0)),
               pl.BlockSpec((RB, H), lambda i: (i, 0))],
    out_shape=[jax.ShapeDtypeStruct((NP, 1), f32),
               jax.ShapeDtypeStruct((NP, H), f32)],
)


def _layer_body(accA_ref, accB_ref, g_ref, dinv_ref, b_ref, w_ref, go_ref):
    dinv = dinv_ref[...]
    h = dinv * (accA_ref[...] + accB_ref[...] + g_ref[...]) + b_ref[...]
    h = jnp.maximum(h, 0.0)
    go_ref[...] = dinv * jnp.dot(h, w_ref[...], preferred_element_type=f32)


_layer = pl.pallas_call(
    _layer_body,
    grid=(NP // RB,),
    in_specs=[pl.BlockSpec((RB, H), lambda i: (i, 0)),
              pl.BlockSpec((RB, H), lambda i: (i, 0)),
              pl.BlockSpec((RB, H), lambda i: (i, 0)),
              pl.BlockSpec((RB, 1), lambda i: (i, 0)),
              pl.BlockSpec((1, H), lambda i: (0, 0)),
              pl.BlockSpec((H, H), lambda i: (0, 0))],
    out_specs=pl.BlockSpec((RB, H), lambda i: (i, 0)),
    out_shape=jax.ShapeDtypeStruct((NP, H), f32),
)


def _pool_body(accA_ref, accB_ref, g_ref, dinv_ref, b_ref, batch_ref,
               wc1_ref, bc1_ref, wc2_ref, bc2_ref, out_ref, seg_acc, cnt_acc):
    i = pl.program_id(0)
    dinv = dinv_ref[...]
    h = dinv * (accA_ref[...] + accB_ref[...] + g_ref[...]) + b_ref[...]
    h = jnp.maximum(h, 0.0)
    row = lax.broadcasted_iota(i32, (RB, 1), 0) + i * RB
    valid = row < N
    seg = lax.broadcasted_iota(i32, (1, G), 1)
    batch_i = batch_ref[...].astype(i32)
    onehot = jnp.where((batch_i == seg) & valid, 1.0, 0.0)
    contrib = lax.dot_general(onehot, h, (((0,), (0,)), ((), ())),
                              preferred_element_type=f32)
    csum = lax.dot_general(onehot, jnp.ones((RB, 1), f32),
                           (((0,), (0,)), ((), ())),
                           preferred_element_type=f32)

    @pl.when(i == 0)
    def _():
        seg_acc[...] = contrib
        cnt_acc[...] = csum
        out_ref[...] = jnp.zeros((G, C), f32)

    @pl.when(i > 0)
    def _():
        seg_acc[...] += contrib
        cnt_acc[...] += csum

    @pl.when(i == NP // RB - 1)
    def _():
        pooled = seg_acc[...] / jnp.maximum(cnt_acc[...], 1.0)
        z = jnp.maximum(
            jnp.dot(pooled, wc1_ref[...], preferred_element_type=f32)
            + bc1_ref[...], 0.0)
        out_ref[...] = (jnp.dot(z, wc2_ref[...], preferred_element_type=f32)
                        + bc2_ref[...])


_pool = pl.pallas_call(
    _pool_body,
    grid=(NP // RB,),
    in_specs=[pl.BlockSpec((RB, H), lambda i: (i, 0)),
              pl.BlockSpec((RB, H), lambda i: (i, 0)),
              pl.BlockSpec((RB, H), lambda i: (i, 0)),
              pl.BlockSpec((RB, 1), lambda i: (i, 0)),
              pl.BlockSpec((1, H), lambda i: (0, 0)),
              pl.BlockSpec((RB, 1), lambda i: (i, 0)),
              pl.BlockSpec((H, 32), lambda i: (0, 0)),
              pl.BlockSpec((1, 32), lambda i: (0, 0)),
              pl.BlockSpec((32, C), lambda i: (0, 0)),
              pl.BlockSpec((1, C), lambda i: (0, 0))],
    out_specs=pl.BlockSpec((G, C), lambda i: (0, 0)),
    out_shape=jax.ShapeDtypeStruct((G, C), f32),
    scratch_shapes=[pltpu.VMEM((G, H), f32), pltpu.VMEM((G, 1), f32)],
)


def kernel(x, edge_index, batch, W1, b1, W2, b2, W3, b3, Wc1, bc1, Wc2, bc2):
    src = edge_index[0].astype(i32)
    dst = edge_index[1].astype(i32)
    pad_k = jnp.arange(EP - E, dtype=i32)
    src_p = jnp.concatenate(
        [src, pad_k % 256]).reshape(TILES * CPT, CHUNK)
    dst_p = jnp.concatenate(
        [dst, N + pad_k % (NP - N)]).reshape(TILES * CPT, CHUNK)
    x_p = jnp.pad(x, ((0, NP - N), (0, 0)))
    batch_p = jnp.pad(batch.astype(f32), (0, NP - N)).reshape(NP, 1)

    deg2 = _deg_call(dst_p)
    h1pre = _mm(x_p, W1)
    dinv, g1 = _dinv_g(deg2[0], deg2[1], h1pre)
    a1 = _msg_call(g1, src_p, dst_p)
    g2 = _layer(a1[0], a1[1], g1, dinv, b1.reshape(1, H), W2)
    a2 = _msg_call(g2, src_p, dst_p)
    g3 = _layer(a2[0], a2[1], g2, dinv, b2.reshape(1, H), W3)
    a3 = _msg_call(g3, src_p, dst_p)
    out = _pool(a3[0], a3[1], g3, dinv, b3.reshape(1, H), batch_p,
                Wc1, bc1.reshape(1, 32), Wc2, bc2.reshape(1, C))
    return out
